# Initial kernel scaffold; baseline (speedup 1.0000x reference)
import jax
import jax.numpy as jnp
from jax.experimental import pallas as pl


def kernel(x, idx):
    raise NotImplementedError("write your pallas kernel here")


# speedup vs baseline: 95.2399x; 95.2399x over previous
import functools

import jax
import jax.numpy as jnp
from jax import lax
from jax.experimental import pallas as pl
from jax.experimental.pallas import tpu as pltpu
from jax.experimental.pallas import tpu_sc as plsc

_C, _T, _BR, _S, _SYN = 64, 16, 8, 16, 32
_NPREV = 8192
_NBR = _C * _T * _BR
_POS = _S * _SYN
_ZSLOT = _NPREV
_TBL = _NPREV + 8

_info = plsc.get_sparse_core_info()
_NC = _info.num_cores
_NW = _NC * _info.num_subcores
_L = 16
_BPW = _NBR // _NW
_GROUPS = _BPW // _L
_CH = _L * _POS

_EMASK = 0x0F0F0F0F
_SEGBIAS = 0x78787878
_BRBIAS = 0x7C7C7C7C
_ONES = 0x01010101


def _sc_body(tbl_hbm, idx_hbm, oe_hbm, oo_hbm,
             tbl_v, ib0, ib1, oe_v, oo_v, sem_t, sem_a, sem_b):
    wid = lax.axis_index("s") * _NC + lax.axis_index("c")
    base = wid * _BPW * _POS
    ct = pltpu.async_copy(tbl_hbm, tbl_v, sem_t)
    bufs = (ib0, ib1)
    sems = (sem_a, sem_b)
    cps = [pltpu.async_copy(idx_hbm.at[pl.ds(base, _CH)], ib0, sem_a), None]
    ct.wait()
    lane = lax.iota(jnp.int32, _L) * _POS
    zero = jnp.zeros((_L,), jnp.int32)
    for g in range(_GROUPS):
        b = g & 1
        if g + 1 < _GROUPS:
            nb = (g + 1) & 1
            cps[nb] = pltpu.async_copy(
                idx_hbm.at[pl.ds(base + (g + 1) * _CH, _CH)], bufs[nb],
                sems[nb])
        cps[b].wait()
        ibuf = bufs[b]

        def seg_body(s, carry, ibuf=ibuf):
            br_e, br_o = carry
            seg_e = zero
            seg_o = zero
            pos0 = s * _SYN
            for k in range(4):
                part = zero
                for j in range(8):
                    iv = lane + (pos0 + (k * 8 + j))
                    raw = plsc.load_gather(ibuf, [iv])
                    safe = jnp.where(raw < 0, _ZSLOT, raw)
                    part = part + plsc.load_gather(tbl_v, [safe])
                seg_e = seg_e + (part & _EMASK)
                seg_o = seg_o + ((part >> 4) & _EMASK)
            br_e = br_e + (((seg_e + _SEGBIAS) >> 7) & _ONES)
            br_o = br_o + (((seg_o + _SEGBIAS) >> 7) & _ONES)
            return br_e, br_o

        br_e, br_o = lax.fori_loop(0, _S, seg_body, (zero, zero))
        oe_v[pl.ds(g * _L, _L)] = ((br_e + _BRBIAS) >> 7) & _ONES
        oo_v[pl.ds(g * _L, _L)] = ((br_o + _BRBIAS) >> 7) & _ONES
    pltpu.sync_copy(oe_v, oe_hbm.at[pl.ds(wid * _BPW, _BPW)])
    pltpu.sync_copy(oo_v, oo_hbm.at[pl.ds(wid * _BPW, _BPW)])


_sc_call = functools.partial(
    pl.kernel,
    mesh=plsc.VectorSubcoreMesh(core_axis_name="c", subcore_axis_name="s"),
    compiler_params=pltpu.CompilerParams(needs_layout_passes=False),
    out_type=[jax.ShapeDtypeStruct((_NBR,), jnp.int32),
              jax.ShapeDtypeStruct((_NBR,), jnp.int32)],
    scratch_types=[
        pltpu.VMEM((_TBL,), jnp.int32),
        pltpu.VMEM((_CH,), jnp.int32),
        pltpu.VMEM((_CH,), jnp.int32),
        pltpu.VMEM((_BPW,), jnp.int32),
        pltpu.VMEM((_BPW,), jnp.int32),
        pltpu.SemaphoreType.DMA,
        pltpu.SemaphoreType.DMA,
        pltpu.SemaphoreType.DMA,
    ],
)(_sc_body)


def kernel(x, idx):
    prev = x != 0
    bits = prev.astype(jnp.int32)
    shifts = (jnp.arange(8, dtype=jnp.int32) * 4)[:, None]
    packed = jnp.sum(bits << shifts, axis=0)
    table = jnp.concatenate(
        [packed, jnp.zeros((_TBL - _NPREV,), jnp.int32)])
    oe, oo = _sc_call(table, idx.reshape(-1))
    rows = []
    for bb in range(8):
        src = oe if bb % 2 == 0 else oo
        rows.append((src >> (8 * (bb // 2))) & 1)
    bon = jnp.stack(rows, axis=0).reshape(8, _C, _T, _BR).astype(jnp.bool_)
    final = bon[:, :, 0].astype(jnp.int32)
    return (final, prev, bon)


# speedup vs baseline: 189.7408x; 1.9922x over previous
import functools

import jax
import jax.numpy as jnp
from jax import lax
from jax.experimental import pallas as pl
from jax.experimental.pallas import tpu as pltpu
from jax.experimental.pallas import tpu_sc as plsc

_C, _T, _BR, _S, _SYN = 64, 16, 8, 16, 32
_NPREV = 8192
_ZSLOT = _NPREV
_TBL = _NPREV + 8
_NPAIR = _T * _BR

_info = plsc.get_sparse_core_info()
_NC = _info.num_cores
_NW = _NC * _info.num_subcores
_L = 16
_PPW = _NPAIR // _NW
_CG = _C // _L
_NBUF = 4

_EMASK = 0x0F0F0F0F
_SEGBIAS = 0x78787878
_BRBIAS = 0x7C7C7C7C
_ONES = 0x01010101


def _sc_body(tbl_hbm, idx_hbm, oe_hbm, oo_hbm,
             tbl_v, seg_v, oe_v, oo_v, sem_t, *sems):
    wid = lax.axis_index("s") * _NC + lax.axis_index("c")
    pair0 = wid * _PPW
    ct = pltpu.async_copy(tbl_hbm, tbl_v, sem_t)

    nplanes = _PPW * _S
    flat0 = pair0 * _S

    def plane_copy(flat, buf):
        t = flat // (_BR * _S)
        br = (flat // _S) % _BR
        s = flat % _S
        return pltpu.async_copy(idx_hbm.at[t, br, s], seg_v.at[buf], sems[buf])

    cps = [None] * _NBUF
    for i in range(_NBUF):
        cps[i] = plane_copy(flat0 + i, i)
    ct.wait()

    zero = jnp.zeros((_L,), jnp.int32)

    def do_plane(buf):
        outs = []
        for cg in range(_CG):
            seg_e = zero
            seg_o = zero
            for k in range(4):
                part = zero
                for j in range(8):
                    raw = seg_v[buf, k * 8 + j, pl.ds(cg * _L, _L)]
                    safe = jnp.where(raw < 0, _ZSLOT, raw)
                    part = part + plsc.load_gather(tbl_v, [safe])
                seg_e = seg_e + (part & _EMASK)
                seg_o = seg_o + ((part >> 4) & _EMASK)
            outs.append((((seg_e + _SEGBIAS) >> 7) & _ONES,
                         ((seg_o + _SEGBIAS) >> 7) & _ONES))
        return outs

    for pair in range(_PPW):
        local0 = pair * _S

        def super_body(si, carry, local0=local0):
            accs = list(carry)
            for b in range(_NBUF):
                local = local0 + si * _NBUF + b
                buf = b
                cps[buf].wait()
                bits = do_plane(buf)
                nxt = local + _NBUF

                @pl.when(nxt < nplanes)
                def _():
                    cp = plane_copy(flat0 + nxt, buf)
                    del cp

                for cg in range(_CG):
                    be, bo = bits[cg]
                    accs[2 * cg] = accs[2 * cg] + be
                    accs[2 * cg + 1] = accs[2 * cg + 1] + bo
            return tuple(accs)

        accs = lax.fori_loop(0, _S // _NBUF, super_body, (zero,) * (2 * _CG))
        for cg in range(_CG):
            off = (pair * _C) + cg * _L
            oe_v[pl.ds(off, _L)] = ((accs[2 * cg] + _BRBIAS) >> 7) & _ONES
            oo_v[pl.ds(off, _L)] = ((accs[2 * cg + 1] + _BRBIAS) >> 7) & _ONES
    out0 = pair0 * _C
    pltpu.sync_copy(oe_v, oe_hbm.at[pl.ds(out0, _PPW * _C)])
    pltpu.sync_copy(oo_v, oo_hbm.at[pl.ds(out0, _PPW * _C)])


_sc_call = functools.partial(
    pl.kernel,
    mesh=plsc.VectorSubcoreMesh(core_axis_name="c", subcore_axis_name="s"),
    compiler_params=pltpu.CompilerParams(needs_layout_passes=False),
    out_type=[jax.ShapeDtypeStruct((_NPAIR * _C,), jnp.int32),
              jax.ShapeDtypeStruct((_NPAIR * _C,), jnp.int32)],
    scratch_types=[
        pltpu.VMEM((_TBL,), jnp.int32),
        pltpu.VMEM((_NBUF, _SYN, _C), jnp.int32),
        pltpu.VMEM((_PPW * _C,), jnp.int32),
        pltpu.VMEM((_PPW * _C,), jnp.int32),
        pltpu.SemaphoreType.DMA,
        pltpu.SemaphoreType.DMA,
        pltpu.SemaphoreType.DMA,
        pltpu.SemaphoreType.DMA,
        pltpu.SemaphoreType.DMA,
    ],
)(_sc_body)


def kernel(x, idx):
    prev = x != 0
    bits = prev.astype(jnp.int32)
    shifts = (jnp.arange(8, dtype=jnp.int32) * 4)[:, None]
    packed = jnp.sum(bits << shifts, axis=0)
    table = jnp.concatenate(
        [packed, jnp.zeros((_TBL - _NPREV,), jnp.int32)])
    idx_t = jnp.transpose(idx, (1, 2, 3, 4, 0))
    oe, oo = _sc_call(table, idx_t)
    oe = oe.reshape(_T, _BR, _C)
    oo = oo.reshape(_T, _BR, _C)
    rows = []
    for bb in range(8):
        src = oe if bb % 2 == 0 else oo
        rows.append((src >> (8 * (bb // 2))) & 1)
    bon = jnp.stack(rows, axis=0)
    bon = jnp.transpose(bon, (0, 3, 1, 2)).astype(jnp.bool_)
    final = bon[:, :, 0].astype(jnp.int32)
    return (final, prev, bon)


# speedup vs baseline: 322.7936x; 1.7012x over previous
import functools

import jax
import jax.numpy as jnp
from jax import lax
from jax.experimental import pallas as pl
from jax.experimental.pallas import tpu as pltpu
from jax.experimental.pallas import tpu_sc as plsc

_C, _T, _BR, _S, _SYN = 64, 16, 8, 16, 32
_NPREV = 8192
_ZSLOT = _NPREV
_TBL = _NPREV + 8
_NPAIR = _T * _BR

_info = plsc.get_sparse_core_info()
_NC = _info.num_cores
_NW = _NC * _info.num_subcores
_L = 16
_PPW = _NPAIR // _NW
_CG = _C // _L
_NBUF = 4

_EMASK = 0x0F0F0F0F
_SEGBIAS = 0x78787878
_BRBIAS = 0x7C7C7C7C
_ONES = 0x01010101


def _sc_body(tbl_hbm, idx_hbm, oe_hbm, oo_hbm,
             tbl_v, seg_v, oe_v, oo_v, sem_t, sem_a, sem_b):
    wid = lax.axis_index("s") * _NC + lax.axis_index("c")
    pair0 = wid * _PPW
    ct = pltpu.async_copy(tbl_hbm, tbl_v, sem_t)
    sems = (sem_a, sem_b)

    _HS = _S // 2

    def blk_copy(blk, buf):
        flat = pair0 + blk // 2
        t = flat // _BR
        br = flat % _BR
        half = blk % 2
        return pltpu.async_copy(
            idx_hbm.at[t, br, pl.ds(half * _HS, _HS)], seg_v.at[buf],
            sems[buf])

    nblk = _PPW * 2
    cps = [blk_copy(0, 0), None]
    ct.wait()

    zero = jnp.zeros((_L,), jnp.int32)
    big = jnp.full((_L,), _ZSLOT, jnp.uint32)

    accs = None
    for blk in range(nblk):
        buf = blk & 1
        if blk + 1 < nblk:
            cps[1 - buf] = blk_copy(blk + 1, 1 - buf)
        cps[buf].wait()
        if blk % 2 == 0:
            accs = (zero,) * (2 * _CG)

        def seg_loop(s, carry, buf=buf):
            accs = list(carry)
            for cg in range(_CG):
                seg_e = zero
                seg_o = zero
                for k in range(4):
                    part = zero
                    for j in range(8):
                        raw = seg_v[buf, s, k * 8 + j, pl.ds(cg * _L, _L)]
                        safe = plsc.bitcast(
                            jnp.minimum(plsc.bitcast(raw, jnp.uint32), big),
                            jnp.int32)
                        part = part + plsc.load_gather(tbl_v, [safe])
                    seg_e = seg_e + (part & _EMASK)
                    seg_o = seg_o + ((part >> 4) & _EMASK)
                accs[2 * cg] = accs[2 * cg] + (
                    ((seg_e + _SEGBIAS) >> 7) & _ONES)
                accs[2 * cg + 1] = accs[2 * cg + 1] + (
                    ((seg_o + _SEGBIAS) >> 7) & _ONES)
            return tuple(accs)

        accs = lax.fori_loop(0, _HS, seg_loop, accs)
        if blk % 2 == 1:
            pair = blk // 2
            for cg in range(_CG):
                off = (pair * _C) + cg * _L
                oe_v[pl.ds(off, _L)] = (
                    (accs[2 * cg] + _BRBIAS) >> 7) & _ONES
                oo_v[pl.ds(off, _L)] = (
                    (accs[2 * cg + 1] + _BRBIAS) >> 7) & _ONES
    out0 = pair0 * _C
    pltpu.sync_copy(oe_v, oe_hbm.at[pl.ds(out0, _PPW * _C)])
    pltpu.sync_copy(oo_v, oo_hbm.at[pl.ds(out0, _PPW * _C)])


_sc_call = functools.partial(
    pl.kernel,
    mesh=plsc.VectorSubcoreMesh(core_axis_name="c", subcore_axis_name="s"),
    compiler_params=pltpu.CompilerParams(needs_layout_passes=False),
    out_type=[jax.ShapeDtypeStruct((_NPAIR * _C,), jnp.int32),
              jax.ShapeDtypeStruct((_NPAIR * _C,), jnp.int32)],
    scratch_types=[
        pltpu.VMEM((_TBL,), jnp.int32),
        pltpu.VMEM((2, _S // 2, _SYN, _C), jnp.int32),
        pltpu.VMEM((_PPW * _C,), jnp.int32),
        pltpu.VMEM((_PPW * _C,), jnp.int32),
        pltpu.SemaphoreType.DMA,
        pltpu.SemaphoreType.DMA,
        pltpu.SemaphoreType.DMA,
    ],
)(_sc_body)


def kernel(x, idx):
    prev = x != 0
    bits = prev.astype(jnp.int32)
    shifts = (jnp.arange(8, dtype=jnp.int32) * 4)[:, None]
    packed = jnp.sum(bits << shifts, axis=0)
    table = jnp.concatenate(
        [packed, jnp.zeros((_TBL - _NPREV,), jnp.int32)])
    idx_t = jnp.transpose(idx, (1, 2, 3, 4, 0))
    oe, oo = _sc_call(table, idx_t)
    oe = oe.reshape(_T, _BR, _C)
    oo = oo.reshape(_T, _BR, _C)
    rows = []
    for bb in range(8):
        src = oe if bb % 2 == 0 else oo
        rows.append((src >> (8 * (bb // 2))) & 1)
    bon = jnp.stack(rows, axis=0)
    bon = jnp.transpose(bon, (0, 3, 1, 2)).astype(jnp.bool_)
    final = bon[:, :, 0].astype(jnp.int32)
    return (final, prev, bon)


# speedup vs baseline: 340.8525x; 1.0559x over previous
import functools

import jax
import jax.numpy as jnp
from jax import lax
from jax.experimental import pallas as pl
from jax.experimental.pallas import tpu as pltpu
from jax.experimental.pallas import tpu_sc as plsc

_C, _T, _BR, _S, _SYN = 64, 16, 8, 16, 32
_NPREV = 8192
_ZSLOT = _NPREV
_TBL = _NPREV + 8
_NPAIR = _T * _BR

_info = plsc.get_sparse_core_info()
_NC = _info.num_cores
_NW = _NC * _info.num_subcores
_L = 16
_PPW = _NPAIR // _NW
_CG = _C // _L
_NBUF = 4

_EMASK = 0x0F0F0F0F
_SEGBIAS = 0x78787878
_BRBIAS = 0x7C7C7C7C
_ONES = 0x01010101


def _sc_body(tbl_hbm, idx_hbm, oe_hbm, oo_hbm,
             tbl_v, seg_v, oe_v, oo_v, sem_t, sem_a, sem_b):
    wid = lax.axis_index("s") * _NC + lax.axis_index("c")
    pair0 = wid * _PPW
    ct = pltpu.async_copy(tbl_hbm, tbl_v, sem_t)
    sems = (sem_a, sem_b)

    _HS = _S // 2

    def blk_copy(blk, buf):
        flat = pair0 + blk // 2
        t = flat // _BR
        br = flat % _BR
        half = blk % 2
        return pltpu.async_copy(
            idx_hbm.at[t, br, pl.ds(half * _HS, _HS)], seg_v.at[buf],
            sems[buf])

    nblk = _PPW * 2
    cps = [blk_copy(0, 0), None]
    ct.wait()

    zero = jnp.zeros((_L,), jnp.int32)
    big = jnp.full((_L,), _ZSLOT, jnp.uint32)

    accs = None
    for blk in range(nblk):
        buf = blk & 1
        if blk + 1 < nblk:
            cps[1 - buf] = blk_copy(blk + 1, 1 - buf)
        cps[buf].wait()
        if blk % 2 == 0:
            accs = [(zero, zero)] * _CG

        for cg in range(_CG):

            def seg_loop(s, carry, buf=buf, cg=cg):
                br_e, br_o = carry
                seg_e = zero
                seg_o = zero
                for k in range(4):
                    part = zero
                    for j in range(8):
                        raw = seg_v[buf, s, k * 8 + j, pl.ds(cg * _L, _L)]
                        safe = plsc.bitcast(
                            jnp.minimum(plsc.bitcast(raw, jnp.uint32), big),
                            jnp.int32)
                        part = part + plsc.load_gather(tbl_v, [safe])
                    seg_e = seg_e + (part & _EMASK)
                    seg_o = seg_o + ((part >> 4) & _EMASK)
                br_e = br_e + (((seg_e + _SEGBIAS) >> 7) & _ONES)
                br_o = br_o + (((seg_o + _SEGBIAS) >> 7) & _ONES)
                return br_e, br_o

            accs[cg] = lax.fori_loop(0, _HS, seg_loop, accs[cg])
        if blk % 2 == 1:
            pair = blk // 2
            for cg in range(_CG):
                br_e, br_o = accs[cg]
                off = (pair * _C) + cg * _L
                oe_v[pl.ds(off, _L)] = ((br_e + _BRBIAS) >> 7) & _ONES
                oo_v[pl.ds(off, _L)] = ((br_o + _BRBIAS) >> 7) & _ONES
    out0 = pair0 * _C
    pltpu.sync_copy(oe_v, oe_hbm.at[pl.ds(out0, _PPW * _C)])
    pltpu.sync_copy(oo_v, oo_hbm.at[pl.ds(out0, _PPW * _C)])


_sc_call = functools.partial(
    pl.kernel,
    mesh=plsc.VectorSubcoreMesh(core_axis_name="c", subcore_axis_name="s"),
    compiler_params=pltpu.CompilerParams(needs_layout_passes=False),
    out_type=[jax.ShapeDtypeStruct((_NPAIR * _C,), jnp.int32),
              jax.ShapeDtypeStruct((_NPAIR * _C,), jnp.int32)],
    scratch_types=[
        pltpu.VMEM((_TBL,), jnp.int32),
        pltpu.VMEM((2, _S // 2, _SYN, _C), jnp.int32),
        pltpu.VMEM((_PPW * _C,), jnp.int32),
        pltpu.VMEM((_PPW * _C,), jnp.int32),
        pltpu.SemaphoreType.DMA,
        pltpu.SemaphoreType.DMA,
        pltpu.SemaphoreType.DMA,
    ],
)(_sc_body)


def kernel(x, idx):
    prev = x != 0
    bits = prev.astype(jnp.int32)
    shifts = (jnp.arange(8, dtype=jnp.int32) * 4)[:, None]
    packed = jnp.sum(bits << shifts, axis=0)
    table = jnp.concatenate(
        [packed, jnp.zeros((_TBL - _NPREV,), jnp.int32)])
    idx_t = jnp.transpose(idx, (1, 2, 3, 4, 0))
    oe, oo = _sc_call(table, idx_t)
    packed_to = jnp.stack([oe, oo], axis=0)
    sh = (jnp.arange(4, dtype=jnp.int32) * 8)[:, None, None]
    bon = (packed_to[None] >> sh) & 1
    bon = bon.reshape(8, _T, _BR, _C)
    bon = jnp.transpose(bon, (0, 3, 1, 2)).astype(jnp.bool_)
    final = bon[:, :, 0].astype(jnp.int32)
    return (final, prev, bon)
